# baseline (speedup 1.0000x reference)
import jax
import jax.numpy as jnp
from jax import lax
from jax.experimental import pallas as pl
from jax.experimental.pallas import tpu as pltpu
from jax.experimental.pallas import tpu_sc as plsc

DIM = 1024
T_ROWS = 8192
NSEG = 8
OUT_ROWS = T_ROWS + NSEG
NW = 32
PERW = T_ROWS // NW
S = 16
NCH = PERW // S
NG = S // 16
NSLOT = 7
PRE = NSLOT - 1


def _take(v, idx):
    dnums = lax.GatherDimensionNumbers(
        offset_dims=(), collapsed_slice_dims=(0,), start_index_map=(0,))
    return lax.gather(v, idx[:, None], dnums, slice_sizes=(1,),
                      mode=lax.GatherScatterMode.PROMISE_IN_BOUNDS)


def _body(flat, w, cu, out, idxs, bufs, wv, cuv, scr, gsems, ssems, ws):
    cid = lax.axis_index("c")
    sid = lax.axis_index("s")
    wid = sid * 2 + cid
    base = wid * PERW

    pltpu.sync_copy(cu.at[pl.ds(0, NSEG)], cuv.at[pl.ds(0, NSEG)])

    lane = lax.iota(jnp.int32, 16)
    cuvec = cuv[...]
    cs = [_take(cuvec, jnp.full((16,), j, jnp.int32)) for j in range(1, NSEG)]

    def compute_oidx(ch):
        cbase = base + ch * S
        ref = idxs[ch % NSLOT]
        for g in range(NG):
            pos = cbase + 16 * g + lane
            seg = jnp.zeros((16,), jnp.int32)
            for v in cs:
                seg = seg + jnp.where(pos >= v, 1, 0)
            ref[pl.ds(16 * g, 16)] = pos + seg + 1

    def gather_start(ch):
        st = pl.multiple_of(base + ch * S, 8)
        pltpu.make_async_copy(flat.at[pl.ds(st, S)], bufs[ch % NSLOT],
                              gsems[ch % NSLOT]).start()

    def gather_wait(ch):
        st = pl.multiple_of(base + ch * S, 8)
        pltpu.make_async_copy(flat.at[pl.ds(st, S)], bufs[ch % NSLOT],
                              gsems[ch % NSLOT]).wait()

    def scatter_start(ch):
        pltpu.make_async_copy(bufs[ch % NSLOT], out.at[idxs[ch % NSLOT]],
                              ssems[ch % NSLOT]).start()

    def scatter_wait(ch):
        pltpu.make_async_copy(bufs[ch % NSLOT], out.at[idxs[ch % NSLOT]],
                              ssems[ch % NSLOT]).wait()

    for ch in range(min(PRE, NCH)):
        compute_oidx(ch)
        gather_start(ch)
    for i in range(NCH):
        gather_wait(i)
        scatter_start(i)
        nxt = i + PRE
        if nxt < NCH:
            if nxt - NSLOT >= 0:
                scatter_wait(nxt - NSLOT)
            compute_oidx(nxt)
            gather_start(nxt)
    for ch in range(max(0, NCH - NSLOT), NCH):
        scatter_wait(ch)

    @pl.when(wid < NSEG)
    def _():
        pltpu.make_async_copy(w, wv, ws).start()
        scr[pl.ds(0, 16)] = _take(cuvec, jnp.full((16,), wid, jnp.int32)) + wid
        f = scr[pl.ds(0, 16)][0]
        pltpu.make_async_copy(w, wv, ws).wait()
        pltpu.sync_copy(wv, out.at[pl.ds(f, 1)])


def _body_flat(flat, w, cu, out, *scratch):
    idxs = list(scratch[0:NSLOT])
    bufs = list(scratch[NSLOT:2 * NSLOT])
    wv, cuv, scr = scratch[2 * NSLOT:2 * NSLOT + 3]
    gsems = list(scratch[2 * NSLOT + 3:3 * NSLOT + 3])
    ssems = list(scratch[3 * NSLOT + 3:4 * NSLOT + 3])
    ws = scratch[4 * NSLOT + 3]
    _body(flat, w, cu, out, idxs, bufs, wv, cuv, scr, gsems, ssems, ws)


def kernel(flat, weight, cu_seqlens):
    mesh = plsc.VectorSubcoreMesh(core_axis_name="c", subcore_axis_name="s")
    scratch = (
        [pltpu.VMEM((S,), jnp.int32)] * NSLOT
        + [pltpu.VMEM((S, DIM), jnp.float32)] * NSLOT
        + [pltpu.VMEM((1, DIM), jnp.float32),
           pltpu.VMEM((16,), jnp.int32),
           pltpu.VMEM((16,), jnp.int32)]
        + [pltpu.SemaphoreType.DMA] * (2 * NSLOT + 1)
    )
    f = pl.kernel(
        _body_flat,
        out_type=jax.ShapeDtypeStruct((OUT_ROWS, DIM), jnp.float32),
        mesh=mesh,
        scratch_types=scratch,
    )
    return f(flat, weight, cu_seqlens)


# speedup vs baseline: 1.0014x; 1.0014x over previous
import jax
import jax.numpy as jnp
from jax import lax
from jax.experimental import pallas as pl
from jax.experimental.pallas import tpu as pltpu
from jax.experimental.pallas import tpu_sc as plsc

DIM = 1024
T_ROWS = 8192
NSEG = 8
OUT_ROWS = T_ROWS + NSEG
NW = 32
PERW = T_ROWS // NW
S = 16
NCH = PERW // S
NG = S // 16
NSLOT = 6
PRE = NSLOT - 1


def _take(v, idx):
    dnums = lax.GatherDimensionNumbers(
        offset_dims=(), collapsed_slice_dims=(0,), start_index_map=(0,))
    return lax.gather(v, idx[:, None], dnums, slice_sizes=(1,),
                      mode=lax.GatherScatterMode.PROMISE_IN_BOUNDS)


def _body(flat, w, cu, out, idxs, bufs, wv, cuv, scr, gsems, ssems, ws):
    cid = lax.axis_index("c")
    sid = lax.axis_index("s")
    wid = sid * 2 + cid
    base = wid * PERW

    pltpu.sync_copy(cu.at[pl.ds(0, NSEG)], cuv.at[pl.ds(0, NSEG)])

    lane = lax.iota(jnp.int32, 16)
    cuvec = cuv[...]
    cs = [_take(cuvec, jnp.full((16,), j, jnp.int32)) for j in range(1, NSEG)]

    def compute_oidx(ch):
        cbase = base + ch * S
        ref = idxs[ch % NSLOT]
        for g in range(NG):
            pos = cbase + 16 * g + lane
            seg = jnp.zeros((16,), jnp.int32)
            for v in cs:
                seg = seg + jnp.where(pos >= v, 1, 0)
            ref[pl.ds(16 * g, 16)] = pos + seg + 1

    def gather_start(ch):
        st = pl.multiple_of(base + ch * S, 8)
        pltpu.make_async_copy(flat.at[pl.ds(st, S)], bufs[ch % NSLOT],
                              gsems[ch % NSLOT]).start()

    def gather_wait(ch):
        st = pl.multiple_of(base + ch * S, 8)
        pltpu.make_async_copy(flat.at[pl.ds(st, S)], bufs[ch % NSLOT],
                              gsems[ch % NSLOT]).wait()

    def scatter_start(ch):
        pltpu.make_async_copy(bufs[ch % NSLOT], out.at[idxs[ch % NSLOT]],
                              ssems[ch % NSLOT]).start()

    def scatter_wait(ch):
        pltpu.make_async_copy(bufs[ch % NSLOT], out.at[idxs[ch % NSLOT]],
                              ssems[ch % NSLOT]).wait()

    for ch in range(min(PRE, NCH)):
        compute_oidx(ch)
        gather_start(ch)
    for i in range(NCH):
        gather_wait(i)
        scatter_start(i)
        nxt = i + PRE
        if nxt < NCH:
            if nxt - NSLOT >= 0:
                scatter_wait(nxt - NSLOT)
            compute_oidx(nxt)
            gather_start(nxt)
    for ch in range(max(0, NCH - NSLOT), NCH):
        scatter_wait(ch)

    @pl.when(wid < NSEG)
    def _():
        pltpu.make_async_copy(w, wv, ws).start()
        scr[pl.ds(0, 16)] = _take(cuvec, jnp.full((16,), wid, jnp.int32)) + wid
        f = scr[pl.ds(0, 16)][0]
        pltpu.make_async_copy(w, wv, ws).wait()
        pltpu.sync_copy(wv, out.at[pl.ds(f, 1)])


def _body_flat(flat, w, cu, out, *scratch):
    idxs = list(scratch[0:NSLOT])
    bufs = list(scratch[NSLOT:2 * NSLOT])
    wv, cuv, scr = scratch[2 * NSLOT:2 * NSLOT + 3]
    gsems = list(scratch[2 * NSLOT + 3:3 * NSLOT + 3])
    ssems = list(scratch[3 * NSLOT + 3:4 * NSLOT + 3])
    ws = scratch[4 * NSLOT + 3]
    _body(flat, w, cu, out, idxs, bufs, wv, cuv, scr, gsems, ssems, ws)


def kernel(flat, weight, cu_seqlens):
    mesh = plsc.VectorSubcoreMesh(core_axis_name="c", subcore_axis_name="s")
    scratch = (
        [pltpu.VMEM((S,), jnp.int32)] * NSLOT
        + [pltpu.VMEM((S, DIM), jnp.float32)] * NSLOT
        + [pltpu.VMEM((1, DIM), jnp.float32),
           pltpu.VMEM((16,), jnp.int32),
           pltpu.VMEM((16,), jnp.int32)]
        + [pltpu.SemaphoreType.DMA] * (2 * NSLOT + 1)
    )
    f = pl.kernel(
        _body_flat,
        out_type=jax.ShapeDtypeStruct((OUT_ROWS, DIM), jnp.float32),
        mesh=mesh,
        scratch_types=scratch,
    )
    return f(flat, weight, cu_seqlens)


# speedup vs baseline: 1.0071x; 1.0057x over previous
import jax
import jax.numpy as jnp
from jax import lax
from jax.experimental import pallas as pl
from jax.experimental.pallas import tpu as pltpu
from jax.experimental.pallas import tpu_sc as plsc

DIM = 1024
T_ROWS = 8192
NSEG = 8
OUT_ROWS = T_ROWS + NSEG
NW = 32
PERW = T_ROWS // NW
S = 16
NCH = PERW // S
NG = S // 16
NSLOT = 6
PRE = NSLOT - 1


def _take(v, idx):
    dnums = lax.GatherDimensionNumbers(
        offset_dims=(), collapsed_slice_dims=(0,), start_index_map=(0,))
    return lax.gather(v, idx[:, None], dnums, slice_sizes=(1,),
                      mode=lax.GatherScatterMode.PROMISE_IN_BOUNDS)


def _body(flat, w, cu, out, idxs, bufs, wv, cuv, scr, gsems, ssems, ws):
    cid = lax.axis_index("c")
    sid = lax.axis_index("s")
    wid = sid * 2 + cid
    base = wid * PERW

    lane = lax.iota(jnp.int32, 16)

    def compute_oidx(ch):
        cbase = base + ch * S
        ref = idxs[ch % NSLOT]
        for g in range(NG):
            pos = cbase + 16 * g + lane
            seg = jnp.zeros((16,), jnp.int32)
            for v in cs:
                seg = seg + jnp.where(pos >= v, 1, 0)
            ref[pl.ds(16 * g, 16)] = pos + seg + 1

    def gather_start(ch):
        st = pl.multiple_of(base + ch * S, 8)
        pltpu.make_async_copy(flat.at[pl.ds(st, S)], bufs[ch % NSLOT],
                              gsems[ch % NSLOT]).start()

    def gather_wait(ch):
        st = pl.multiple_of(base + ch * S, 8)
        pltpu.make_async_copy(flat.at[pl.ds(st, S)], bufs[ch % NSLOT],
                              gsems[ch % NSLOT]).wait()

    def scatter_start(ch):
        pltpu.make_async_copy(bufs[ch % NSLOT], out.at[idxs[ch % NSLOT]],
                              ssems[ch % NSLOT]).start()

    def scatter_wait(ch):
        pltpu.make_async_copy(bufs[ch % NSLOT], out.at[idxs[ch % NSLOT]],
                              ssems[ch % NSLOT]).wait()

    for ch in range(min(PRE, NCH)):
        gather_start(ch)

    pltpu.sync_copy(cu.at[pl.ds(0, NSEG)], cuv.at[pl.ds(0, NSEG)])
    cuvec = cuv[...]
    cs = [_take(cuvec, jnp.full((16,), j, jnp.int32)) for j in range(1, NSEG)]

    for ch in range(min(PRE, NCH)):
        compute_oidx(ch)
    for i in range(NCH):
        gather_wait(i)
        scatter_start(i)
        nxt = i + PRE
        if nxt < NCH:
            if nxt - NSLOT >= 0:
                scatter_wait(nxt - NSLOT)
            compute_oidx(nxt)
            gather_start(nxt)
    for ch in range(max(0, NCH - NSLOT), NCH):
        scatter_wait(ch)

    @pl.when(wid < NSEG)
    def _():
        pltpu.make_async_copy(w, wv, ws).start()
        scr[pl.ds(0, 16)] = _take(cuvec, jnp.full((16,), wid, jnp.int32)) + wid
        f = scr[pl.ds(0, 16)][0]
        pltpu.make_async_copy(w, wv, ws).wait()
        pltpu.sync_copy(wv, out.at[pl.ds(f, 1)])


def _body_flat(flat, w, cu, out, *scratch):
    idxs = list(scratch[0:NSLOT])
    bufs = list(scratch[NSLOT:2 * NSLOT])
    wv, cuv, scr = scratch[2 * NSLOT:2 * NSLOT + 3]
    gsems = list(scratch[2 * NSLOT + 3:3 * NSLOT + 3])
    ssems = list(scratch[3 * NSLOT + 3:4 * NSLOT + 3])
    ws = scratch[4 * NSLOT + 3]
    _body(flat, w, cu, out, idxs, bufs, wv, cuv, scr, gsems, ssems, ws)


def kernel(flat, weight, cu_seqlens):
    mesh = plsc.VectorSubcoreMesh(core_axis_name="c", subcore_axis_name="s")
    scratch = (
        [pltpu.VMEM((S,), jnp.int32)] * NSLOT
        + [pltpu.VMEM((S, DIM), jnp.float32)] * NSLOT
        + [pltpu.VMEM((1, DIM), jnp.float32),
           pltpu.VMEM((16,), jnp.int32),
           pltpu.VMEM((16,), jnp.int32)]
        + [pltpu.SemaphoreType.DMA] * (2 * NSLOT + 1)
    )
    f = pl.kernel(
        _body_flat,
        out_type=jax.ShapeDtypeStruct((OUT_ROWS, DIM), jnp.float32),
        mesh=mesh,
        scratch_types=scratch,
    )
    return f(flat, weight, cu_seqlens)
